# baseline (speedup 1.0000x reference)
import functools

import jax
import jax.numpy as jnp
from jax import lax
from jax.experimental import pallas as pl
from jax.experimental.pallas import tpu as pltpu
from jax.experimental.pallas import tpu_sc as plsc

SEQ = 200
BATCH = 1024
EMBED = 64

NC = 2
NS = 16
NW = NC * NS
CHUNK = 128
CPR = BATCH // CHUNK
NCHUNK = SEQ * CPR // NW
ST = SEQ // 8

K = 5
NG = NCHUNK // K

_mesh = plsc.VectorSubcoreMesh(core_axis_name="c", subcore_axis_name="s")


@functools.partial(
    pl.kernel,
    mesh=_mesh,
    out_type=jax.ShapeDtypeStruct((SEQ, BATCH, EMBED), jnp.float32),
    compiler_params=pltpu.CompilerParams(use_tc_tiling_on_sc=False),
    scratch_types=[
        pltpu.VMEM((2, CPR, 8, CHUNK), jnp.int32),
        pltpu.VMEM((K, CHUNK, EMBED), jnp.float32),
        pltpu.VMEM((K, CHUNK, EMBED), jnp.float32),
        pltpu.SemaphoreType.DMA,
        pltpu.SemaphoreType.DMA,
        pltpu.SemaphoreType.DMA,
        pltpu.SemaphoreType.DMA,
    ],
)
def _gather_kernel(idx_hbm, table_hbm, out_hbm, idx_v, buf_a, buf_b,
                   gsem_a, gsem_b, wsem_a, wsem_b):
    wid = lax.axis_index("s") * NC + lax.axis_index("c")
    chunk0 = wid * NCHUNK
    st0 = jnp.minimum((chunk0 // CPR) // 8, ST - 2)
    pltpu.sync_copy(idx_hbm.at[pl.ds(st0, 2)], idx_v)

    def chunk_coords(c):
        s = c // CPR
        bt = c % CPR
        return s, bt, s // 8 - st0, s % 8

    def idx_slice(c):
        _, bt, lst, si = chunk_coords(c)
        return idx_v.at[lst, bt, si]

    def out_slice(c):
        s, bt, _, _ = chunk_coords(c)
        return out_hbm.at[s, pl.ds(bt * CHUNK, CHUNK)]

    def fire_gathers(g, buf, gsem):
        for b in range(K):
            c = chunk0 + g * K + b
            pltpu.async_copy(table_hbm.at[idx_slice(c)], buf.at[b], gsem)

    def drain_then_write(g, buf, gsem, wsem):
        for b in range(K):
            c = chunk0 + g * K + b
            pltpu.make_async_copy(table_hbm.at[idx_slice(c)], buf.at[b],
                                  gsem).wait()
        for b in range(K):
            c = chunk0 + g * K + b
            pltpu.async_copy(buf.at[b], out_slice(c), wsem)

    def drain_writes(g, buf, wsem):
        for b in range(K):
            c = chunk0 + g * K + b
            pltpu.make_async_copy(buf.at[b], out_slice(c), wsem).wait()

    fire_gathers(0, buf_a, gsem_a)

    def body(i, carry):
        g0 = 2 * i
        fire_gathers(g0 + 1, buf_b, gsem_b)
        drain_then_write(g0, buf_a, gsem_a, wsem_a)
        drain_writes(g0, buf_a, wsem_a)

        @pl.when(g0 + 2 < NG)
        def _():
            fire_gathers(g0 + 2, buf_a, gsem_a)

        drain_then_write(g0 + 1, buf_b, gsem_b, wsem_b)
        drain_writes(g0 + 1, buf_b, wsem_b)
        return carry

    lax.fori_loop(0, NG // 2, body, 0)


def kernel(inputs, inputs_len, table):
    del inputs_len
    idx4 = inputs.reshape(ST, 8, CPR, CHUNK).transpose(0, 2, 1, 3)
    return _gather_kernel(idx4, table)
